# Initial kernel scaffold; baseline (speedup 1.0000x reference)
import jax
import jax.numpy as jnp
from jax.experimental import pallas as pl


def kernel(x, edge_index, W1, b1, W2, b2, conv_w, conv_b, fc_w, fc_b):
    raise NotImplementedError("write your pallas kernel here")


# speedup vs baseline: 21.7331x; 21.7331x over previous
import functools

import jax
import jax.numpy as jnp
from jax import lax
from jax.experimental import pallas as pl
from jax.experimental.pallas import tpu as pltpu
from jax.experimental.pallas import tpu_sc as plsc

_NC = 2
_NS = 16
_NW = _NC * _NS


def _sc_degree(dst, zeros_n):
    n = zeros_n.shape[0]
    e = dst.shape[0]
    ept = e // _NW
    ch = 80
    nch = ept // ch
    mesh = plsc.VectorSubcoreMesh(core_axis_name="c", subcore_axis_name="s")

    @functools.partial(
        pl.kernel,
        out_type=jax.ShapeDtypeStruct((_NW, n), jnp.float32),
        mesh=mesh,
        scratch_types=[
            pltpu.VMEM((ch,), jnp.int32),
            pltpu.VMEM((n,), jnp.float32),
        ],
        compiler_params=pltpu.CompilerParams(needs_layout_passes=False,
                                             use_tc_tiling_on_sc=False),
    )
    def k(dst_hbm, z_hbm, out_hbm, dstv, hist):
        c = lax.axis_index("c")
        s = lax.axis_index("s")
        wid = c * _NS + s
        pltpu.sync_copy(z_hbm, hist)
        e0 = wid * ept
        ones = jnp.full((16,), 1.0, jnp.float32)

        def body(i, carry):
            pltpu.sync_copy(dst_hbm.at[pl.ds(e0 + i * ch, ch)], dstv)
            for j in range(ch // 16):
                idx = dstv[pl.ds(j * 16, 16)]
                plsc.addupdate_scatter(hist, [idx], ones)
            return carry

        lax.fori_loop(0, nch, body, 0)
        pltpu.sync_copy(hist, out_hbm.at[wid])

    return k(dst, zeros_n)


def _sc_scatter(g, src, dst, zeros_nf):
    n, f = g.shape
    e = src.shape[0]
    ept = e // _NW
    ch = 80
    nch = ept // ch
    rpt = n // _NS
    mesh = plsc.VectorSubcoreMesh(core_axis_name="c", subcore_axis_name="s")

    @functools.partial(
        pl.kernel,
        out_type=jax.ShapeDtypeStruct((_NC, n, f), jnp.float32),
        mesh=mesh,
        scratch_types=[
            pltpu.VMEM((ch,), jnp.int32),
            pltpu.VMEM((ch,), jnp.int32),
            pltpu.VMEM((ch, f), jnp.float32),
            pltpu.VMEM_SHARED((n, f), jnp.float32),
            pltpu.VMEM_SHARED((n, f), jnp.float32),
            pltpu.SemaphoreType.DMA,
        ],
        compiler_params=pltpu.CompilerParams(needs_layout_passes=False,
                                             use_tc_tiling_on_sc=False),
    )
    def k(g_hbm, src_hbm, dst_hbm, z_hbm, out_hbm,
          srcv, dstv, rows, g_sh, acc_sh, sem):
        c = lax.axis_index("c")
        s = lax.axis_index("s")
        wid = c * _NS + s
        r0 = s * rpt
        pltpu.sync_copy(g_hbm.at[pl.ds(r0, rpt)], g_sh.at[pl.ds(r0, rpt)])
        pltpu.sync_copy(z_hbm.at[pl.ds(r0, rpt)], acc_sh.at[pl.ds(r0, rpt)])
        plsc.subcore_barrier()
        e0 = wid * ept

        def body(i, carry):
            base = e0 + i * ch
            pltpu.sync_copy(src_hbm.at[pl.ds(base, ch)], srcv)
            pltpu.sync_copy(dst_hbm.at[pl.ds(base, ch)], dstv)
            pltpu.async_copy(g_sh.at[srcv], rows, sem).wait()
            pltpu.sync_copy(rows, acc_sh.at[dstv], add=True)
            return carry

        lax.fori_loop(0, nch, body, 0)
        plsc.subcore_barrier()
        pltpu.sync_copy(acc_sh.at[pl.ds(r0, rpt)],
                        out_hbm.at[c, pl.ds(r0, rpt)])

    return k(g, src, dst, zeros_nf)


def _tc_dinv(hist):
    def body(h_ref, o_ref):
        deg = 1.0 + jnp.sum(h_ref[...], axis=0, keepdims=True)
        o_ref[...] = lax.rsqrt(deg)

    return pl.pallas_call(
        body,
        out_shape=jax.ShapeDtypeStruct((1, hist.shape[1]), jnp.float32),
    )(hist)


def _tc_g1(x, w1, dinv_col):
    n = x.shape[0]
    f = w1.shape[1]

    def body(x_ref, w_ref, d_ref, o_ref):
        h = jnp.dot(x_ref[...], w_ref[...],
                    preferred_element_type=jnp.float32)
        o_ref[...] = h * d_ref[...]

    return pl.pallas_call(
        body,
        out_shape=jax.ShapeDtypeStruct((n, f), jnp.float32),
    )(x, w1, dinv_col)


def _tc_g2(s1, g1, dinv_col, b1):
    n, f = g1.shape

    def body(s_ref, g_ref, d_ref, b_ref, o_ref):
        d = d_ref[...]
        m = d * (s_ref[0] + s_ref[1] + g_ref[...]) + b_ref[...]
        o_ref[...] = jnp.maximum(m, 0.0) * d

    return pl.pallas_call(
        body,
        out_shape=jax.ShapeDtypeStruct((n, f), jnp.float32),
    )(s1, g1, dinv_col, b1.reshape(1, f))


def _tc_final(s2, g2, dinv_col, w2, b2, k0, k1, k2, fc_wt, fc_b):
    n = g2.shape[0]
    ncls = fc_wt.shape[1]

    def body(s_ref, g_ref, d_ref, w2_ref, b2_ref, k0_ref, k1_ref, k2_ref,
             fw_ref, fb_ref, o_ref):
        m = d_ref[...] * (s_ref[0] + s_ref[1] + g_ref[...])
        h2 = jnp.dot(m, w2_ref[...],
                     preferred_element_type=jnp.float32) + b2_ref[...]
        a = jnp.dot(h2, k0_ref[...], preferred_element_type=jnp.float32)
        b = jnp.dot(h2, k1_ref[...], preferred_element_type=jnp.float32)
        c = jnp.dot(h2, k2_ref[...], preferred_element_type=jnp.float32)
        row = lax.broadcasted_iota(jnp.int32, a.shape, 0)
        a_dn = jnp.where(row == 0, 0.0, pltpu.roll(a, 1, 0))
        c_up = jnp.where(row == n - 1, 0.0, pltpu.roll(c, n - 1, 0))
        y = b + a_dn + c_up
        p = jnp.max(y, axis=0, keepdims=True)
        o_ref[...] = jnp.dot(p, fw_ref[...],
                             preferred_element_type=jnp.float32) + fb_ref[...]

    return pl.pallas_call(
        body,
        out_shape=jax.ShapeDtypeStruct((1, ncls), jnp.float32),
    )(s2, g2, dinv_col, w2, b2.reshape(1, -1), k0, k1, k2,
      fc_wt, fc_b.reshape(1, -1))


def kernel(x, edge_index, W1, b1, W2, b2, conv_w, conv_b, fc_w, fc_b):
    n = x.shape[0]
    f = W1.shape[1]
    src = edge_index[0]
    dst = edge_index[1]

    zeros_n = jnp.zeros((n,), jnp.float32)
    zeros_nf = jnp.zeros((n, f), jnp.float32)

    hist = _sc_degree(dst, zeros_n)
    dinv_row = _tc_dinv(hist)
    dinv_col = dinv_row.reshape(n, 1)

    g1 = _tc_g1(x, W1, dinv_col)
    s1 = _sc_scatter(g1, src, dst, zeros_nf)
    g2 = _tc_g2(s1, g1, dinv_col, b1)
    s2 = _sc_scatter(g2, src, dst, zeros_nf)

    k0 = conv_w[:, :, 0, 0].T
    k1 = conv_w[:, :, 0, 1].T
    k2 = conv_w[:, :, 0, 2].T
    y_bias = conv_b.reshape(1, -1)

    out = _tc_final(s2, g2, dinv_col, W2, b2, k0, k1, k2,
                    fc_w.T, fc_b + y_bias @ fc_w.T)
    return out


# speedup vs baseline: 60.4389x; 2.7810x over previous
import functools

import jax
import jax.numpy as jnp
from jax import lax
from jax.experimental import pallas as pl
from jax.experimental.pallas import tpu as pltpu
from jax.experimental.pallas import tpu_sc as plsc

_NC = 2
_NS = 16
_NW = _NC * _NS


_CH = 80


def _sc_degree(dst3, zeros_n):
    n = zeros_n.shape[0]
    nch, ch = dst3.shape[1:]
    mesh = plsc.VectorSubcoreMesh(core_axis_name="c", subcore_axis_name="s")

    @functools.partial(
        pl.kernel,
        out_type=jax.ShapeDtypeStruct((_NW, n), jnp.float32),
        mesh=mesh,
        scratch_types=[
            pltpu.VMEM((nch, ch), jnp.int32),
            pltpu.VMEM((n,), jnp.float32),
        ],
        compiler_params=pltpu.CompilerParams(needs_layout_passes=False,
                                             use_tc_tiling_on_sc=False),
    )
    def k(dst_hbm, z_hbm, out_hbm, dst_all, hist):
        c = lax.axis_index("c")
        s = lax.axis_index("s")
        wid = c * _NS + s
        pltpu.sync_copy(dst_hbm.at[wid], dst_all)
        pltpu.sync_copy(z_hbm, hist)
        ones = jnp.full((16,), 1.0, jnp.float32)

        def body(i, carry):
            for j in range(ch // 16):
                idx = dst_all[i, pl.ds(j * 16, 16)]
                plsc.addupdate_scatter(hist, [idx], ones)
            return carry

        lax.fori_loop(0, nch, body, 0)
        pltpu.sync_copy(hist, out_hbm.at[wid])

    return k(dst3, zeros_n)


def _sc_scatter(g, src3, dst3, zeros_nf):
    n, f = g.shape
    nch, ch = src3.shape[1:]
    npair = (nch - 1) // 2
    rpt = n // _NS
    mesh = plsc.VectorSubcoreMesh(core_axis_name="c", subcore_axis_name="s")

    @functools.partial(
        pl.kernel,
        out_type=jax.ShapeDtypeStruct((_NC, n, f), jnp.float32),
        mesh=mesh,
        scratch_types=[
            pltpu.VMEM((nch, ch), jnp.int32),
            pltpu.VMEM((nch, ch), jnp.int32),
            pltpu.VMEM((ch, f), jnp.float32),
            pltpu.VMEM((ch, f), jnp.float32),
            pltpu.VMEM_SHARED((n, f), jnp.float32),
            pltpu.VMEM_SHARED((n, f), jnp.float32),
            pltpu.SemaphoreType.DMA,
            pltpu.SemaphoreType.DMA,
        ],
        compiler_params=pltpu.CompilerParams(needs_layout_passes=False,
                                             use_tc_tiling_on_sc=False),
    )
    def k(g_hbm, src_hbm, dst_hbm, z_hbm, out_hbm,
          src_all, dst_all, rows0, rows1, g_sh, acc_sh, sem0, sem1):
        c = lax.axis_index("c")
        s = lax.axis_index("s")
        wid = c * _NS + s
        r0 = s * rpt
        pltpu.sync_copy(src_hbm.at[wid], src_all)
        pltpu.sync_copy(dst_hbm.at[wid], dst_all)
        pltpu.sync_copy(g_hbm.at[pl.ds(r0, rpt)], g_sh.at[pl.ds(r0, rpt)])
        pltpu.sync_copy(z_hbm.at[pl.ds(r0, rpt)], acc_sh.at[pl.ds(r0, rpt)])
        plsc.subcore_barrier()

        pltpu.async_copy(g_sh.at[src_all.at[0]], rows0, sem0)

        def pair(j, carry):
            b1 = 2 * j + 1
            cp1 = pltpu.async_copy(g_sh.at[src_all.at[b1]], rows1, sem1)
            pltpu.make_async_copy(g_sh.at[src_all.at[b1 - 1]], rows0,
                                  sem0).wait()
            pltpu.sync_copy(rows0, acc_sh.at[dst_all.at[b1 - 1]], add=True)
            pltpu.async_copy(g_sh.at[src_all.at[b1 + 1]], rows0, sem0)
            cp1.wait()
            pltpu.sync_copy(rows1, acc_sh.at[dst_all.at[b1]], add=True)
            return carry

        lax.fori_loop(0, npair, pair, 0)
        pltpu.make_async_copy(g_sh.at[src_all.at[nch - 1]], rows0, sem0).wait()
        pltpu.sync_copy(rows0, acc_sh.at[dst_all.at[nch - 1]], add=True)

        plsc.subcore_barrier()
        pltpu.sync_copy(acc_sh.at[pl.ds(r0, rpt)],
                        out_hbm.at[c, pl.ds(r0, rpt)])

    return k(g, src3, dst3, zeros_nf)


def _tc_dinv(hist):
    def body(h_ref, o_ref):
        deg = 1.0 + jnp.sum(h_ref[...], axis=0, keepdims=True)
        o_ref[...] = lax.rsqrt(deg)

    return pl.pallas_call(
        body,
        out_shape=jax.ShapeDtypeStruct((1, hist.shape[1]), jnp.float32),
    )(hist)


def _tc_g1(x, w1, dinv_col):
    n = x.shape[0]
    f = w1.shape[1]

    def body(x_ref, w_ref, d_ref, o_ref):
        h = jnp.dot(x_ref[...], w_ref[...],
                    preferred_element_type=jnp.float32)
        o_ref[...] = h * d_ref[...]

    return pl.pallas_call(
        body,
        out_shape=jax.ShapeDtypeStruct((n, f), jnp.float32),
    )(x, w1, dinv_col)


def _tc_g2(s1, g1, dinv_col, b1):
    n, f = g1.shape

    def body(s_ref, g_ref, d_ref, b_ref, o_ref):
        d = d_ref[...]
        m = d * (s_ref[0] + s_ref[1] + g_ref[...]) + b_ref[...]
        o_ref[...] = jnp.maximum(m, 0.0) * d

    return pl.pallas_call(
        body,
        out_shape=jax.ShapeDtypeStruct((n, f), jnp.float32),
    )(s1, g1, dinv_col, b1.reshape(1, f))


def _tc_final(s2, g2, dinv_col, w2, b2, k0, k1, k2, fc_wt, fc_b):
    n = g2.shape[0]
    ncls = fc_wt.shape[1]

    def body(s_ref, g_ref, d_ref, w2_ref, b2_ref, k0_ref, k1_ref, k2_ref,
             fw_ref, fb_ref, o_ref):
        m = d_ref[...] * (s_ref[0] + s_ref[1] + g_ref[...])
        h2 = jnp.dot(m, w2_ref[...],
                     preferred_element_type=jnp.float32) + b2_ref[...]
        a = jnp.dot(h2, k0_ref[...], preferred_element_type=jnp.float32)
        b = jnp.dot(h2, k1_ref[...], preferred_element_type=jnp.float32)
        c = jnp.dot(h2, k2_ref[...], preferred_element_type=jnp.float32)
        row = lax.broadcasted_iota(jnp.int32, a.shape, 0)
        a_dn = jnp.where(row == 0, 0.0, pltpu.roll(a, 1, 0))
        c_up = jnp.where(row == n - 1, 0.0, pltpu.roll(c, n - 1, 0))
        y = b + a_dn + c_up
        p = jnp.max(y, axis=0, keepdims=True)
        o_ref[...] = jnp.dot(p, fw_ref[...],
                             preferred_element_type=jnp.float32) + fb_ref[...]

    return pl.pallas_call(
        body,
        out_shape=jax.ShapeDtypeStruct((1, ncls), jnp.float32),
    )(s2, g2, dinv_col, w2, b2.reshape(1, -1), k0, k1, k2,
      fc_wt, fc_b.reshape(1, -1))


def kernel(x, edge_index, W1, b1, W2, b2, conv_w, conv_b, fc_w, fc_b):
    n = x.shape[0]
    f = W1.shape[1]
    e = edge_index.shape[1]
    ept = e // _NW
    nch = ept // _CH
    assert ept % _CH == 0 and nch % 2 == 1
    src3 = edge_index[0].reshape(_NW, nch, _CH)
    dst3 = edge_index[1].reshape(_NW, nch, _CH)

    zeros_n = jnp.zeros((n,), jnp.float32)
    zeros_nf = jnp.zeros((n, f), jnp.float32)

    hist = _sc_degree(dst3, zeros_n)
    dinv_row = _tc_dinv(hist)
    dinv_col = dinv_row.reshape(n, 1)

    g1 = _tc_g1(x, W1, dinv_col)
    s1 = _sc_scatter(g1, src3, dst3, zeros_nf)
    g2 = _tc_g2(s1, g1, dinv_col, b1)
    s2 = _sc_scatter(g2, src3, dst3, zeros_nf)

    k0 = conv_w[:, :, 0, 0].T
    k1 = conv_w[:, :, 0, 1].T
    k2 = conv_w[:, :, 0, 2].T
    y_bias = conv_b.reshape(1, -1)

    out = _tc_final(s2, g2, dinv_col, W2, b2, k0, k1, k2,
                    fc_w.T, fc_b + y_bias @ fc_w.T)
    return out


# speedup vs baseline: 66.8741x; 1.1065x over previous
import functools

import jax
import jax.numpy as jnp
from jax import lax
from jax.experimental import pallas as pl
from jax.experimental.pallas import tpu as pltpu
from jax.experimental.pallas import tpu_sc as plsc

_NC = 2
_NS = 16
_NW = _NC * _NS


_CH = 80


def _sc_degree(e4, zeros_n):
    n = zeros_n.shape[0]
    nch, ch = e4.shape[2:]
    mesh = plsc.VectorSubcoreMesh(core_axis_name="c", subcore_axis_name="s")

    @functools.partial(
        pl.kernel,
        out_type=jax.ShapeDtypeStruct((_NW, n), jnp.float32),
        mesh=mesh,
        scratch_types=[
            pltpu.VMEM((nch, ch), jnp.int32),
            pltpu.VMEM((n,), jnp.float32),
        ],
        compiler_params=pltpu.CompilerParams(needs_layout_passes=False,
                                             use_tc_tiling_on_sc=False),
    )
    def k(e_hbm, z_hbm, out_hbm, dst_all, hist):
        c = lax.axis_index("c")
        s = lax.axis_index("s")
        wid = c * _NS + s
        pltpu.sync_copy(e_hbm.at[1, wid], dst_all)
        pltpu.sync_copy(z_hbm, hist)
        ones = jnp.full((16,), 1.0, jnp.float32)

        def body(i, carry):
            for j in range(ch // 16):
                idx = dst_all[i, pl.ds(j * 16, 16)]
                plsc.addupdate_scatter(hist, [idx], ones)
            return carry

        lax.fori_loop(0, nch, body, 0)
        pltpu.sync_copy(hist, out_hbm.at[wid])

    return k(e4, zeros_n)


def _sc_scatter(g, e4, zeros_nf):
    n, f = g.shape
    nch, ch = e4.shape[2:]
    npair = (nch - 1) // 2
    rpt = n // _NS
    mesh = plsc.VectorSubcoreMesh(core_axis_name="c", subcore_axis_name="s")

    @functools.partial(
        pl.kernel,
        out_type=jax.ShapeDtypeStruct((_NC, n, f), jnp.float32),
        mesh=mesh,
        scratch_types=[
            pltpu.VMEM((nch, ch), jnp.int32),
            pltpu.VMEM((nch, ch), jnp.int32),
            pltpu.VMEM((ch, f), jnp.float32),
            pltpu.VMEM((ch, f), jnp.float32),
            pltpu.VMEM_SHARED((n, f), jnp.float32),
            pltpu.VMEM_SHARED((n, f), jnp.float32),
            pltpu.SemaphoreType.DMA,
            pltpu.SemaphoreType.DMA,
        ],
        compiler_params=pltpu.CompilerParams(needs_layout_passes=False,
                                             use_tc_tiling_on_sc=False),
    )
    def k(g_hbm, e_hbm, z_hbm, out_hbm,
          src_all, dst_all, rows0, rows1, g_sh, acc_sh, sem0, sem1):
        c = lax.axis_index("c")
        s = lax.axis_index("s")
        wid = c * _NS + s
        r0 = s * rpt
        pltpu.sync_copy(e_hbm.at[0, wid], src_all)
        pltpu.sync_copy(e_hbm.at[1, wid], dst_all)
        pltpu.sync_copy(g_hbm.at[pl.ds(r0, rpt)], g_sh.at[pl.ds(r0, rpt)])
        pltpu.sync_copy(z_hbm.at[pl.ds(r0, rpt)], acc_sh.at[pl.ds(r0, rpt)])
        plsc.subcore_barrier()

        pltpu.async_copy(g_sh.at[src_all.at[0]], rows0, sem0)

        def pair(j, carry):
            b1 = 2 * j + 1
            cp1 = pltpu.async_copy(g_sh.at[src_all.at[b1]], rows1, sem1)
            pltpu.make_async_copy(g_sh.at[src_all.at[b1 - 1]], rows0,
                                  sem0).wait()
            pltpu.sync_copy(rows0, acc_sh.at[dst_all.at[b1 - 1]], add=True)
            pltpu.async_copy(g_sh.at[src_all.at[b1 + 1]], rows0, sem0)
            cp1.wait()
            pltpu.sync_copy(rows1, acc_sh.at[dst_all.at[b1]], add=True)
            return carry

        lax.fori_loop(0, npair, pair, 0)
        pltpu.make_async_copy(g_sh.at[src_all.at[nch - 1]], rows0, sem0).wait()
        pltpu.sync_copy(rows0, acc_sh.at[dst_all.at[nch - 1]], add=True)

        plsc.subcore_barrier()
        pltpu.sync_copy(acc_sh.at[pl.ds(r0, rpt)],
                        out_hbm.at[c, pl.ds(r0, rpt)])

    return k(g, e4, zeros_nf)


def _tc_h(x, w1):
    n = x.shape[0]
    f = w1.shape[1]

    def body(x_ref, w_ref, o_ref):
        o_ref[...] = jnp.dot(x_ref[...], w_ref[...],
                             preferred_element_type=jnp.float32)

    return pl.pallas_call(
        body,
        out_shape=jax.ShapeDtypeStruct((n, f), jnp.float32),
    )(x, w1)


def _tc_dinv_g1(hist, h):
    n, f = h.shape

    def body(hi_ref, h_ref, d_ref, g_ref):
        deg = 1.0 + jnp.sum(hi_ref[...], axis=0, keepdims=True)
        dcol = lax.rsqrt(deg).reshape(n, 1)
        d_ref[...] = dcol
        g_ref[...] = h_ref[...] * dcol

    return pl.pallas_call(
        body,
        out_shape=(jax.ShapeDtypeStruct((n, 1), jnp.float32),
                   jax.ShapeDtypeStruct((n, f), jnp.float32)),
    )(hist, h)


def _tc_g2(s1, g1, dinv_col, b1):
    n, f = g1.shape

    def body(s_ref, g_ref, d_ref, b_ref, o_ref):
        d = d_ref[...]
        m = d * (s_ref[0] + s_ref[1] + g_ref[...]) + b_ref[...]
        o_ref[...] = jnp.maximum(m, 0.0) * d

    return pl.pallas_call(
        body,
        out_shape=jax.ShapeDtypeStruct((n, f), jnp.float32),
    )(s1, g1, dinv_col, b1.reshape(1, f))


def _tc_final(s2, g2, dinv_col, w2, b2, k0, k1, k2, fc_wt, fc_b):
    n = g2.shape[0]
    ncls = fc_wt.shape[1]

    def body(s_ref, g_ref, d_ref, w2_ref, b2_ref, k0_ref, k1_ref, k2_ref,
             fw_ref, fb_ref, o_ref):
        m = d_ref[...] * (s_ref[0] + s_ref[1] + g_ref[...])
        h2 = jnp.dot(m, w2_ref[...],
                     preferred_element_type=jnp.float32) + b2_ref[...]
        a = jnp.dot(h2, k0_ref[...], preferred_element_type=jnp.float32)
        b = jnp.dot(h2, k1_ref[...], preferred_element_type=jnp.float32)
        c = jnp.dot(h2, k2_ref[...], preferred_element_type=jnp.float32)
        row = lax.broadcasted_iota(jnp.int32, a.shape, 0)
        a_dn = jnp.where(row == 0, 0.0, pltpu.roll(a, 1, 0))
        c_up = jnp.where(row == n - 1, 0.0, pltpu.roll(c, n - 1, 0))
        y = b + a_dn + c_up
        p = jnp.max(y, axis=0, keepdims=True)
        o_ref[...] = jnp.dot(p, fw_ref[...],
                             preferred_element_type=jnp.float32) + fb_ref[...]

    return pl.pallas_call(
        body,
        out_shape=jax.ShapeDtypeStruct((1, ncls), jnp.float32),
    )(s2, g2, dinv_col, w2, b2.reshape(1, -1), k0, k1, k2,
      fc_wt, fc_b.reshape(1, -1))


def kernel(x, edge_index, W1, b1, W2, b2, conv_w, conv_b, fc_w, fc_b):
    n = x.shape[0]
    f = W1.shape[1]
    e = edge_index.shape[1]
    ept = e // _NW
    nch = ept // _CH
    assert ept % _CH == 0 and nch % 2 == 1
    e4 = edge_index.reshape(2, _NW, nch, _CH)

    zeros_n = jnp.zeros((n,), jnp.float32)
    zeros_nf = jnp.zeros((n, f), jnp.float32)

    h = _tc_h(x, W1)
    hist = _sc_degree(e4, zeros_n)
    dinv_col, g1 = _tc_dinv_g1(hist, h)
    s1 = _sc_scatter(g1, e4, zeros_nf)
    g2 = _tc_g2(s1, g1, dinv_col, b1)
    s2 = _sc_scatter(g2, e4, zeros_nf)

    k0 = conv_w[:, :, 0, 0].T
    k1 = conv_w[:, :, 0, 1].T
    k2 = conv_w[:, :, 0, 2].T
    y_bias = conv_b.reshape(1, -1)

    out = _tc_final(s2, g2, dinv_col, W2, b2, k0, k1, k2,
                    fc_w.T, fc_b + y_bias @ fc_w.T)
    return out
